# baseline (speedup 1.0000x reference)
import functools

import jax
import jax.numpy as jnp
from jax import lax
from jax.experimental import pallas as pl
from jax.experimental.pallas import tpu as pltpu
from jax.experimental.pallas import tpu_sc as plsc

_EMBED = 768
_SEQ = 2048
_BATCH = 16
_NKEYS = 100
_KEYS_PAD = 104
_TOPK = 3

_TAIL = 768
_HEAD = _SEQ - _TAIL
_NTILES = 32
_SPLIT = 2
_ROWS_PER_TILE = _TAIL // _SPLIT
_CHUNK_ROWS = 64
_NCHUNKS = _ROWS_PER_TILE // _CHUNK_ROWS
_VPR = _EMBED // 16


def _tc_head_body(x_ref, out_ref):
    c = pl.program_id(0)
    out_ref[pl.ds(c, 1), :] = jnp.sum(x_ref[0], axis=0, keepdims=True)


def _tc_head_sums(x):
    return pl.pallas_call(
        _tc_head_body,
        grid=(_BATCH,),
        in_specs=[pl.BlockSpec((1, _HEAD, _EMBED), lambda c: (c, 0, 0))],
        out_specs=pl.BlockSpec((_BATCH, _EMBED), lambda c: (0, 0)),
        out_shape=jax.ShapeDtypeStruct((_BATCH, _EMBED), jnp.float32),
        compiler_params=pltpu.CompilerParams(
            dimension_semantics=("arbitrary",)),
    )(x)


def _sc_tail_sums(x_flat):
    mesh = plsc.VectorSubcoreMesh(core_axis_name="c", subcore_axis_name="s")
    nc = plsc.get_sparse_core_info().num_cores
    chunk_words = _CHUNK_ROWS * _EMBED

    @functools.partial(
        pl.kernel,
        mesh=mesh,
        out_type=jax.ShapeDtypeStruct((_SPLIT, _BATCH, _EMBED), jnp.float32),
        scratch_types=[
            pltpu.VMEM((chunk_words,), jnp.float32),
            pltpu.VMEM((chunk_words,), jnp.float32),
            pltpu.VMEM((_EMBED,), jnp.float32),
            pltpu.SemaphoreType.DMA,
            pltpu.SemaphoreType.DMA,
        ],
    )
    def run(x_hbm, st_hbm, buf0, buf1, acc_v, sem0, sem1):
        wid = lax.axis_index("s") * nc + lax.axis_index("c")
        b = wid // _SPLIT
        h = wid % _SPLIT
        base = (b * _SEQ + _HEAD + h * _ROWS_PER_TILE) * _EMBED

        bufs = (buf0, buf1)
        sems = (sem0, sem1)

        def start(g):
            return pltpu.async_copy(
                x_hbm.at[pl.ds(base + g * chunk_words, chunk_words)],
                bufs[g % 2], sems[g % 2])

        def accum(buf, accs, lo):
            def body(r, carry):
                off = r * _EMBED
                return tuple(
                    carry[j] + buf[pl.ds(off + (lo + j) * 16, 16)]
                    for j in range(len(carry)))
            return lax.fori_loop(0, _CHUNK_ROWS, body, accs)

        half = _VPR // 2
        zero = jnp.zeros((16,), jnp.float32)
        accs_lo = (zero,) * half
        accs_hi = (zero,) * half

        cp = start(0)
        for g in range(_NCHUNKS):
            cp.wait()
            if g + 1 < _NCHUNKS:
                cp = start(g + 1)
            buf = bufs[g % 2]
            accs_lo = accum(buf, accs_lo, 0)
            accs_hi = accum(buf, accs_hi, half)

        for j in range(half):
            acc_v[pl.ds(j * 16, 16)] = accs_lo[j]
            acc_v[pl.ds((half + j) * 16, 16)] = accs_hi[j]
        pltpu.sync_copy(acc_v, st_hbm.at[h, b])

    return run(x_flat)


def _tc_merge_body(pk_ref, sh_ref, st_ref, sim_ref, tv_ref, ti_ref, red_ref):
    total = sh_ref[...] + st_ref[0] + st_ref[1]
    xm = total * (1.0 / _SEQ)
    ss = jnp.sum(xm * xm, axis=-1, keepdims=True)
    xn = xm * lax.rsqrt(jnp.maximum(ss, 1e-12))
    pk = pk_ref[...]
    ps = jnp.sum(pk * pk, axis=-1, keepdims=True)
    pkn = pk * lax.rsqrt(jnp.maximum(ps, 1e-12))
    sim = lax.dot_general(xn, pkn, (((1,), (1,)), ((), ())),
                          preferred_element_type=jnp.float32)
    kiota = lax.broadcasted_iota(jnp.int32, (_BATCH, _KEYS_PAD), 1)
    s = jnp.where(kiota < _NKEYS, sim, -3.0)
    sim_ref[...] = s[:, :_NKEYS]

    vals = []
    idxs = []
    for _ in range(_TOPK):
        m = jnp.max(s, axis=1, keepdims=True)
        i = jnp.min(jnp.where(s == m, kiota, _KEYS_PAD), axis=1,
                    keepdims=True)
        vals.append(m)
        idxs.append(i)
        s = jnp.where(kiota == i, -4.0, s)
    tv = jnp.concatenate(vals, axis=1)
    tv_ref[...] = tv
    ti_ref[...] = jnp.concatenate(idxs, axis=1)
    red_ref[...] = jnp.sum(tv, keepdims=True).reshape(1, 1) * (1.0 / _BATCH)


def _tc_merge(prompt_key, sh, st):
    return pl.pallas_call(
        _tc_merge_body,
        grid=(1,),
        in_specs=[
            pl.BlockSpec((_KEYS_PAD, _EMBED), lambda c: (0, 0)),
            pl.BlockSpec((_BATCH, _EMBED), lambda c: (0, 0)),
            pl.BlockSpec((_SPLIT, _BATCH, _EMBED), lambda c: (0, 0, 0)),
        ],
        out_specs=[
            pl.BlockSpec((_BATCH, _NKEYS), lambda c: (0, 0)),
            pl.BlockSpec((_BATCH, _TOPK), lambda c: (0, 0)),
            pl.BlockSpec((_BATCH, _TOPK), lambda c: (0, 0)),
            pl.BlockSpec((1, 1), lambda c: (0, 0)),
        ],
        out_shape=[
            jax.ShapeDtypeStruct((_BATCH, _NKEYS), jnp.float32),
            jax.ShapeDtypeStruct((_BATCH, _TOPK), jnp.float32),
            jax.ShapeDtypeStruct((_BATCH, _TOPK), jnp.int32),
            jax.ShapeDtypeStruct((1, 1), jnp.float32),
        ],
    )(prompt_key, sh, st)


def kernel(x_embed, y, task_id, prompt_key):
    st = _sc_tail_sums(x_embed.reshape(-1))
    sh = _tc_head_sums(x_embed)
    sim, topk_sim, topk_idx, red = _tc_merge(prompt_key, sh, st)
    return (sim, topk_sim, topk_idx, red.reshape(()))


# speedup vs baseline: 2.6746x; 2.6746x over previous
import functools

import jax
import jax.numpy as jnp
from jax import lax
from jax.experimental import pallas as pl
from jax.experimental.pallas import tpu as pltpu
from jax.experimental.pallas import tpu_sc as plsc

_EMBED = 768
_SEQ = 2048
_BATCH = 16
_NKEYS = 100
_KEYS_PAD = 104
_TOPK = 3

_TAIL = 768
_HEAD = _SEQ - _TAIL
_NTILES = 32
_SPLIT = 2
_ROWS_PER_TILE = _TAIL // _SPLIT
_CHUNK_ROWS = 64
_NCHUNKS = _ROWS_PER_TILE // _CHUNK_ROWS
_VPR = _EMBED // 16


def _tc_head_body(x_ref, out_ref):
    c = pl.program_id(0)
    out_ref[pl.ds(c, 1), :] = jnp.sum(x_ref[0], axis=0, keepdims=True)


def _tc_head_sums(x):
    return pl.pallas_call(
        _tc_head_body,
        grid=(_BATCH,),
        in_specs=[pl.BlockSpec((1, _HEAD, _EMBED), lambda c: (c, 0, 0))],
        out_specs=pl.BlockSpec((_BATCH, _EMBED), lambda c: (0, 0)),
        out_shape=jax.ShapeDtypeStruct((_BATCH, _EMBED), jnp.float32),
        compiler_params=pltpu.CompilerParams(
            dimension_semantics=("arbitrary",)),
    )(x)


def _sc_tail_sums(x):
    mesh = plsc.VectorSubcoreMesh(core_axis_name="c", subcore_axis_name="s")
    nc = plsc.get_sparse_core_info().num_cores

    @functools.partial(
        pl.kernel,
        mesh=mesh,
        out_type=jax.ShapeDtypeStruct((_SPLIT, _BATCH, _EMBED), jnp.float32),
        scratch_types=[
            pltpu.VMEM((_CHUNK_ROWS, _EMBED), jnp.float32),
            pltpu.VMEM((_CHUNK_ROWS, _EMBED), jnp.float32),
            pltpu.VMEM((_EMBED,), jnp.float32),
            pltpu.SemaphoreType.DMA,
            pltpu.SemaphoreType.DMA,
        ],
    )
    def run(x_hbm, st_hbm, buf0, buf1, acc_v, sem0, sem1):
        wid = lax.axis_index("s") * nc + lax.axis_index("c")
        b = wid // _SPLIT
        h = wid % _SPLIT
        row0 = _HEAD + h * _ROWS_PER_TILE

        bufs = (buf0, buf1)
        sems = (sem0, sem1)

        def start(g):
            return pltpu.async_copy(
                x_hbm.at[b, pl.ds(row0 + g * _CHUNK_ROWS, _CHUNK_ROWS), :],
                bufs[g % 2], sems[g % 2])

        def accum(buf, accs, lo):
            def body(r, carry):
                return tuple(
                    carry[j] + buf[r, pl.ds((lo + j) * 16, 16)]
                    for j in range(len(carry)))
            return lax.fori_loop(0, _CHUNK_ROWS, body, accs)

        half = _VPR // 2
        zero = jnp.zeros((16,), jnp.float32)
        accs_lo = (zero,) * half
        accs_hi = (zero,) * half

        cp = start(0)
        for g in range(_NCHUNKS):
            cp.wait()
            if g + 1 < _NCHUNKS:
                cp = start(g + 1)
            buf = bufs[g % 2]
            accs_lo = accum(buf, accs_lo, 0)
            accs_hi = accum(buf, accs_hi, half)

        for j in range(half):
            acc_v[pl.ds(j * 16, 16)] = accs_lo[j]
            acc_v[pl.ds((half + j) * 16, 16)] = accs_hi[j]
        pltpu.sync_copy(acc_v, st_hbm.at[h, b])

    return run(x)


def _tc_merge_body(pk_ref, sh_ref, st_ref, sim_ref, tv_ref, ti_ref, red_ref):
    total = sh_ref[...] + st_ref[0] + st_ref[1]
    xm = total * (1.0 / _SEQ)
    ss = jnp.sum(xm * xm, axis=-1, keepdims=True)
    xn = xm * lax.rsqrt(jnp.maximum(ss, 1e-12))
    pk = pk_ref[...]
    ps = jnp.sum(pk * pk, axis=-1, keepdims=True)
    pkn = pk * lax.rsqrt(jnp.maximum(ps, 1e-12))
    sim = lax.dot_general(xn, pkn, (((1,), (1,)), ((), ())),
                          preferred_element_type=jnp.float32)
    kiota = lax.broadcasted_iota(jnp.int32, (_BATCH, _KEYS_PAD), 1)
    s = jnp.where(kiota < _NKEYS, sim, -3.0)
    sim_ref[...] = s[:, :_NKEYS]

    vals = []
    idxs = []
    for _ in range(_TOPK):
        m = jnp.max(s, axis=1, keepdims=True)
        i = jnp.min(jnp.where(s == m, kiota, _KEYS_PAD), axis=1,
                    keepdims=True)
        vals.append(m)
        idxs.append(i)
        s = jnp.where(kiota == i, -4.0, s)
    tv = jnp.concatenate(vals, axis=1)
    tv_ref[...] = tv
    ti_ref[...] = jnp.concatenate(idxs, axis=1)
    red_ref[...] = jnp.sum(tv, keepdims=True).reshape(1, 1) * (1.0 / _BATCH)


def _tc_merge(prompt_key, sh, st):
    return pl.pallas_call(
        _tc_merge_body,
        grid=(1,),
        in_specs=[
            pl.BlockSpec((_KEYS_PAD, _EMBED), lambda c: (0, 0)),
            pl.BlockSpec((_BATCH, _EMBED), lambda c: (0, 0)),
            pl.BlockSpec((_SPLIT, _BATCH, _EMBED), lambda c: (0, 0, 0)),
        ],
        out_specs=[
            pl.BlockSpec((_BATCH, _NKEYS), lambda c: (0, 0)),
            pl.BlockSpec((_BATCH, _TOPK), lambda c: (0, 0)),
            pl.BlockSpec((_BATCH, _TOPK), lambda c: (0, 0)),
            pl.BlockSpec((1, 1), lambda c: (0, 0)),
        ],
        out_shape=[
            jax.ShapeDtypeStruct((_BATCH, _NKEYS), jnp.float32),
            jax.ShapeDtypeStruct((_BATCH, _TOPK), jnp.float32),
            jax.ShapeDtypeStruct((_BATCH, _TOPK), jnp.int32),
            jax.ShapeDtypeStruct((1, 1), jnp.float32),
        ],
    )(prompt_key, sh, st)


def kernel(x_embed, y, task_id, prompt_key):
    st = _sc_tail_sums(x_embed)
    sh = _tc_head_sums(x_embed)
    sim, topk_sim, topk_idx, red = _tc_merge(prompt_key, sh, st)
    return (sim, topk_sim, topk_idx, red.reshape(()))


# speedup vs baseline: 2.7589x; 1.0315x over previous
import functools

import jax
import jax.numpy as jnp
from jax import lax
from jax.experimental import pallas as pl
from jax.experimental.pallas import tpu as pltpu
from jax.experimental.pallas import tpu_sc as plsc

_EMBED = 768
_SEQ = 2048
_BATCH = 16
_NKEYS = 100
_KEYS_PAD = 104
_TOPK = 3

_TAIL = 384
_HEAD = _SEQ - _TAIL
_NTILES = 32
_SPLIT = 2
_ROWS_PER_TILE = _TAIL // _SPLIT
_CHUNK_ROWS = 64
_NCHUNKS = _ROWS_PER_TILE // _CHUNK_ROWS
_VPR = _EMBED // 16


def _tc_head_body(x_ref, out_ref):
    c = pl.program_id(0)
    out_ref[pl.ds(c, 1), :] = jnp.sum(x_ref[0], axis=0, keepdims=True)


def _tc_head_sums(x):
    return pl.pallas_call(
        _tc_head_body,
        grid=(_BATCH,),
        in_specs=[pl.BlockSpec((1, _HEAD, _EMBED), lambda c: (c, 0, 0))],
        out_specs=pl.BlockSpec((_BATCH, _EMBED), lambda c: (0, 0)),
        out_shape=jax.ShapeDtypeStruct((_BATCH, _EMBED), jnp.float32),
        compiler_params=pltpu.CompilerParams(
            dimension_semantics=("arbitrary",)),
    )(x)


def _sc_tail_sums(x):
    mesh = plsc.VectorSubcoreMesh(core_axis_name="c", subcore_axis_name="s")
    nc = plsc.get_sparse_core_info().num_cores

    @functools.partial(
        pl.kernel,
        mesh=mesh,
        out_type=jax.ShapeDtypeStruct((_SPLIT, _BATCH, _EMBED), jnp.float32),
        scratch_types=[
            pltpu.VMEM((_CHUNK_ROWS, _EMBED), jnp.float32),
            pltpu.VMEM((_CHUNK_ROWS, _EMBED), jnp.float32),
            pltpu.VMEM((_EMBED,), jnp.float32),
            pltpu.SemaphoreType.DMA,
            pltpu.SemaphoreType.DMA,
        ],
    )
    def run(x_hbm, st_hbm, buf0, buf1, acc_v, sem0, sem1):
        wid = lax.axis_index("s") * nc + lax.axis_index("c")
        b = wid // _SPLIT
        h = wid % _SPLIT
        row0 = _HEAD + h * _ROWS_PER_TILE

        bufs = (buf0, buf1)
        sems = (sem0, sem1)

        def start(g):
            return pltpu.async_copy(
                x_hbm.at[b, pl.ds(row0 + g * _CHUNK_ROWS, _CHUNK_ROWS), :],
                bufs[g % 2], sems[g % 2])

        def accum(buf, accs, lo):
            def body(r, carry):
                return tuple(
                    carry[j] + buf[r, pl.ds((lo + j) * 16, 16)]
                    for j in range(len(carry)))
            return lax.fori_loop(0, _CHUNK_ROWS, body, accs)

        half = _VPR // 2
        zero = jnp.zeros((16,), jnp.float32)
        accs_lo = (zero,) * half
        accs_hi = (zero,) * half

        cp = start(0)
        for g in range(_NCHUNKS):
            cp.wait()
            if g + 1 < _NCHUNKS:
                cp = start(g + 1)
            buf = bufs[g % 2]
            accs_lo = accum(buf, accs_lo, 0)
            accs_hi = accum(buf, accs_hi, half)

        for j in range(half):
            acc_v[pl.ds(j * 16, 16)] = accs_lo[j]
            acc_v[pl.ds((half + j) * 16, 16)] = accs_hi[j]
        pltpu.sync_copy(acc_v, st_hbm.at[h, b])

    return run(x)


def _tc_merge_body(pk_ref, sh_ref, st_ref, sim_ref, tv_ref, ti_ref, red_ref):
    total = sh_ref[...] + st_ref[0] + st_ref[1]
    xm = total * (1.0 / _SEQ)
    ss = jnp.sum(xm * xm, axis=-1, keepdims=True)
    xn = xm * lax.rsqrt(jnp.maximum(ss, 1e-12))
    pk = pk_ref[...]
    ps = jnp.sum(pk * pk, axis=-1, keepdims=True)
    pkn = pk * lax.rsqrt(jnp.maximum(ps, 1e-12))
    sim = lax.dot_general(xn, pkn, (((1,), (1,)), ((), ())),
                          preferred_element_type=jnp.float32)
    kiota = lax.broadcasted_iota(jnp.int32, (_BATCH, _KEYS_PAD), 1)
    s = jnp.where(kiota < _NKEYS, sim, -3.0)
    sim_ref[...] = s[:, :_NKEYS]

    vals = []
    idxs = []
    for _ in range(_TOPK):
        m = jnp.max(s, axis=1, keepdims=True)
        i = jnp.min(jnp.where(s == m, kiota, _KEYS_PAD), axis=1,
                    keepdims=True)
        vals.append(m)
        idxs.append(i)
        s = jnp.where(kiota == i, -4.0, s)
    tv = jnp.concatenate(vals, axis=1)
    tv_ref[...] = tv
    ti_ref[...] = jnp.concatenate(idxs, axis=1)
    red_ref[...] = jnp.sum(tv, keepdims=True).reshape(1, 1) * (1.0 / _BATCH)


def _tc_merge(prompt_key, sh, st):
    return pl.pallas_call(
        _tc_merge_body,
        grid=(1,),
        in_specs=[
            pl.BlockSpec((_KEYS_PAD, _EMBED), lambda c: (0, 0)),
            pl.BlockSpec((_BATCH, _EMBED), lambda c: (0, 0)),
            pl.BlockSpec((_SPLIT, _BATCH, _EMBED), lambda c: (0, 0, 0)),
        ],
        out_specs=[
            pl.BlockSpec((_BATCH, _NKEYS), lambda c: (0, 0)),
            pl.BlockSpec((_BATCH, _TOPK), lambda c: (0, 0)),
            pl.BlockSpec((_BATCH, _TOPK), lambda c: (0, 0)),
            pl.BlockSpec((1, 1), lambda c: (0, 0)),
        ],
        out_shape=[
            jax.ShapeDtypeStruct((_BATCH, _NKEYS), jnp.float32),
            jax.ShapeDtypeStruct((_BATCH, _TOPK), jnp.float32),
            jax.ShapeDtypeStruct((_BATCH, _TOPK), jnp.int32),
            jax.ShapeDtypeStruct((1, 1), jnp.float32),
        ],
    )(prompt_key, sh, st)


def kernel(x_embed, y, task_id, prompt_key):
    st = _sc_tail_sums(x_embed)
    sh = _tc_head_sums(x_embed)
    sim, topk_sim, topk_idx, red = _tc_merge(prompt_key, sh, st)
    return (sim, topk_sim, topk_idx, red.reshape(()))


# speedup vs baseline: 2.8079x; 1.0177x over previous
import functools

import jax
import jax.numpy as jnp
from jax import lax
from jax.experimental import pallas as pl
from jax.experimental.pallas import tpu as pltpu
from jax.experimental.pallas import tpu_sc as plsc

_EMBED = 768
_SEQ = 2048
_BATCH = 16
_NKEYS = 100
_KEYS_PAD = 104
_TOPK = 3

_TAIL = 128
_HEAD = _SEQ - _TAIL
_NTILES = 32
_SPLIT = 2
_ROWS_PER_TILE = _TAIL // _SPLIT
_CHUNK_ROWS = 64
_NCHUNKS = _ROWS_PER_TILE // _CHUNK_ROWS
_VPR = _EMBED // 16


def _tc_head_body(x_ref, out_ref):
    c = pl.program_id(0)
    out_ref[pl.ds(c, 1), :] = jnp.sum(x_ref[0], axis=0, keepdims=True)


def _tc_head_sums(x):
    return pl.pallas_call(
        _tc_head_body,
        grid=(_BATCH,),
        in_specs=[pl.BlockSpec((1, _HEAD, _EMBED), lambda c: (c, 0, 0))],
        out_specs=pl.BlockSpec((_BATCH, _EMBED), lambda c: (0, 0)),
        out_shape=jax.ShapeDtypeStruct((_BATCH, _EMBED), jnp.float32),
        compiler_params=pltpu.CompilerParams(
            dimension_semantics=("arbitrary",)),
    )(x)


def _sc_tail_sums(x):
    mesh = plsc.VectorSubcoreMesh(core_axis_name="c", subcore_axis_name="s")
    nc = plsc.get_sparse_core_info().num_cores

    @functools.partial(
        pl.kernel,
        mesh=mesh,
        out_type=jax.ShapeDtypeStruct((_SPLIT, _BATCH, _EMBED), jnp.float32),
        scratch_types=[
            pltpu.VMEM((_CHUNK_ROWS, _EMBED), jnp.float32),
            pltpu.VMEM((_CHUNK_ROWS, _EMBED), jnp.float32),
            pltpu.VMEM((_EMBED,), jnp.float32),
            pltpu.SemaphoreType.DMA,
            pltpu.SemaphoreType.DMA,
        ],
    )
    def run(x_hbm, st_hbm, buf0, buf1, acc_v, sem0, sem1):
        wid = lax.axis_index("s") * nc + lax.axis_index("c")
        b = wid // _SPLIT
        h = wid % _SPLIT
        row0 = _HEAD + h * _ROWS_PER_TILE

        bufs = (buf0, buf1)
        sems = (sem0, sem1)

        def start(g):
            return pltpu.async_copy(
                x_hbm.at[b, pl.ds(row0 + g * _CHUNK_ROWS, _CHUNK_ROWS), :],
                bufs[g % 2], sems[g % 2])

        def accum(buf, accs, lo):
            def body(r, carry):
                return tuple(
                    carry[j] + buf[r, pl.ds((lo + j) * 16, 16)]
                    for j in range(len(carry)))
            return lax.fori_loop(0, _CHUNK_ROWS, body, accs)

        half = _VPR // 2
        zero = jnp.zeros((16,), jnp.float32)
        accs_lo = (zero,) * half
        accs_hi = (zero,) * half

        cp = start(0)
        for g in range(_NCHUNKS):
            cp.wait()
            if g + 1 < _NCHUNKS:
                cp = start(g + 1)
            buf = bufs[g % 2]
            accs_lo = accum(buf, accs_lo, 0)
            accs_hi = accum(buf, accs_hi, half)

        for j in range(half):
            acc_v[pl.ds(j * 16, 16)] = accs_lo[j]
            acc_v[pl.ds((half + j) * 16, 16)] = accs_hi[j]
        pltpu.sync_copy(acc_v, st_hbm.at[h, b])

    return run(x)


def _tc_merge_body(pk_ref, sh_ref, st_ref, sim_ref, tv_ref, ti_ref, red_ref):
    total = sh_ref[...] + st_ref[0] + st_ref[1]
    xm = total * (1.0 / _SEQ)
    ss = jnp.sum(xm * xm, axis=-1, keepdims=True)
    xn = xm * lax.rsqrt(jnp.maximum(ss, 1e-12))
    pk = pk_ref[...]
    ps = jnp.sum(pk * pk, axis=-1, keepdims=True)
    pkn = pk * lax.rsqrt(jnp.maximum(ps, 1e-12))
    sim = lax.dot_general(xn, pkn, (((1,), (1,)), ((), ())),
                          preferred_element_type=jnp.float32)
    kiota = lax.broadcasted_iota(jnp.int32, (_BATCH, _KEYS_PAD), 1)
    s = jnp.where(kiota < _NKEYS, sim, -3.0)
    sim_ref[...] = s[:, :_NKEYS]

    vals = []
    idxs = []
    for _ in range(_TOPK):
        m = jnp.max(s, axis=1, keepdims=True)
        i = jnp.min(jnp.where(s == m, kiota, _KEYS_PAD), axis=1,
                    keepdims=True)
        vals.append(m)
        idxs.append(i)
        s = jnp.where(kiota == i, -4.0, s)
    tv = jnp.concatenate(vals, axis=1)
    tv_ref[...] = tv
    ti_ref[...] = jnp.concatenate(idxs, axis=1)
    red_ref[...] = jnp.sum(tv, keepdims=True).reshape(1, 1) * (1.0 / _BATCH)


def _tc_merge(prompt_key, sh, st):
    return pl.pallas_call(
        _tc_merge_body,
        grid=(1,),
        in_specs=[
            pl.BlockSpec((_KEYS_PAD, _EMBED), lambda c: (0, 0)),
            pl.BlockSpec((_BATCH, _EMBED), lambda c: (0, 0)),
            pl.BlockSpec((_SPLIT, _BATCH, _EMBED), lambda c: (0, 0, 0)),
        ],
        out_specs=[
            pl.BlockSpec((_BATCH, _NKEYS), lambda c: (0, 0)),
            pl.BlockSpec((_BATCH, _TOPK), lambda c: (0, 0)),
            pl.BlockSpec((_BATCH, _TOPK), lambda c: (0, 0)),
            pl.BlockSpec((1, 1), lambda c: (0, 0)),
        ],
        out_shape=[
            jax.ShapeDtypeStruct((_BATCH, _NKEYS), jnp.float32),
            jax.ShapeDtypeStruct((_BATCH, _TOPK), jnp.float32),
            jax.ShapeDtypeStruct((_BATCH, _TOPK), jnp.int32),
            jax.ShapeDtypeStruct((1, 1), jnp.float32),
        ],
    )(prompt_key, sh, st)


def kernel(x_embed, y, task_id, prompt_key):
    st = _sc_tail_sums(x_embed)
    sh = _tc_head_sums(x_embed)
    sim, topk_sim, topk_idx, red = _tc_merge(prompt_key, sh, st)
    return (sim, topk_sim, topk_idx, red.reshape(()))


# speedup vs baseline: 2.8353x; 1.0098x over previous
import functools

import jax
import jax.numpy as jnp
from jax import lax
from jax.experimental import pallas as pl
from jax.experimental.pallas import tpu as pltpu
from jax.experimental.pallas import tpu_sc as plsc

_EMBED = 768
_SEQ = 2048
_BATCH = 16
_NKEYS = 100
_KEYS_PAD = 104
_TOPK = 3

_TAIL = 64
_HEAD = _SEQ - _TAIL
_NTILES = 32
_SPLIT = 2
_ROWS_PER_TILE = _TAIL // _SPLIT
_CHUNK_ROWS = min(64, _ROWS_PER_TILE)
_NCHUNKS = _ROWS_PER_TILE // _CHUNK_ROWS
_VPR = _EMBED // 16


def _tc_head_body(x_ref, out_ref):
    c = pl.program_id(0)
    out_ref[pl.ds(c, 1), :] = jnp.sum(x_ref[0], axis=0, keepdims=True)


def _tc_head_sums(x):
    return pl.pallas_call(
        _tc_head_body,
        grid=(_BATCH,),
        in_specs=[pl.BlockSpec((1, _HEAD, _EMBED), lambda c: (c, 0, 0))],
        out_specs=pl.BlockSpec((_BATCH, _EMBED), lambda c: (0, 0)),
        out_shape=jax.ShapeDtypeStruct((_BATCH, _EMBED), jnp.float32),
        compiler_params=pltpu.CompilerParams(
            dimension_semantics=("arbitrary",)),
    )(x)


def _sc_tail_sums(x):
    mesh = plsc.VectorSubcoreMesh(core_axis_name="c", subcore_axis_name="s")
    nc = plsc.get_sparse_core_info().num_cores

    @functools.partial(
        pl.kernel,
        mesh=mesh,
        out_type=jax.ShapeDtypeStruct((_SPLIT, _BATCH, _EMBED), jnp.float32),
        scratch_types=[
            pltpu.VMEM((_CHUNK_ROWS, _EMBED), jnp.float32),
            pltpu.VMEM((_CHUNK_ROWS, _EMBED), jnp.float32),
            pltpu.VMEM((_EMBED,), jnp.float32),
            pltpu.SemaphoreType.DMA,
            pltpu.SemaphoreType.DMA,
        ],
    )
    def run(x_hbm, st_hbm, buf0, buf1, acc_v, sem0, sem1):
        wid = lax.axis_index("s") * nc + lax.axis_index("c")
        b = wid // _SPLIT
        h = wid % _SPLIT
        row0 = _HEAD + h * _ROWS_PER_TILE

        bufs = (buf0, buf1)
        sems = (sem0, sem1)

        def start(g):
            return pltpu.async_copy(
                x_hbm.at[b, pl.ds(row0 + g * _CHUNK_ROWS, _CHUNK_ROWS), :],
                bufs[g % 2], sems[g % 2])

        def accum(buf, accs, lo):
            def body(r, carry):
                return tuple(
                    carry[j] + buf[r, pl.ds((lo + j) * 16, 16)]
                    for j in range(len(carry)))
            return lax.fori_loop(0, _CHUNK_ROWS, body, accs)

        half = _VPR // 2
        zero = jnp.zeros((16,), jnp.float32)
        accs_lo = (zero,) * half
        accs_hi = (zero,) * half

        cp = start(0)
        for g in range(_NCHUNKS):
            cp.wait()
            if g + 1 < _NCHUNKS:
                cp = start(g + 1)
            buf = bufs[g % 2]
            accs_lo = accum(buf, accs_lo, 0)
            accs_hi = accum(buf, accs_hi, half)

        for j in range(half):
            acc_v[pl.ds(j * 16, 16)] = accs_lo[j]
            acc_v[pl.ds((half + j) * 16, 16)] = accs_hi[j]
        pltpu.sync_copy(acc_v, st_hbm.at[h, b])

    return run(x)


def _tc_merge_body(pk_ref, sh_ref, st_ref, sim_ref, tv_ref, ti_ref, red_ref):
    total = sh_ref[...] + st_ref[0] + st_ref[1]
    xm = total * (1.0 / _SEQ)
    ss = jnp.sum(xm * xm, axis=-1, keepdims=True)
    xn = xm * lax.rsqrt(jnp.maximum(ss, 1e-12))
    pk = pk_ref[...]
    ps = jnp.sum(pk * pk, axis=-1, keepdims=True)
    pkn = pk * lax.rsqrt(jnp.maximum(ps, 1e-12))
    sim = lax.dot_general(xn, pkn, (((1,), (1,)), ((), ())),
                          preferred_element_type=jnp.float32)
    kiota = lax.broadcasted_iota(jnp.int32, (_BATCH, _KEYS_PAD), 1)
    s = jnp.where(kiota < _NKEYS, sim, -3.0)
    sim_ref[...] = s[:, :_NKEYS]

    vals = []
    idxs = []
    for _ in range(_TOPK):
        m = jnp.max(s, axis=1, keepdims=True)
        i = jnp.min(jnp.where(s == m, kiota, _KEYS_PAD), axis=1,
                    keepdims=True)
        vals.append(m)
        idxs.append(i)
        s = jnp.where(kiota == i, -4.0, s)
    tv = jnp.concatenate(vals, axis=1)
    tv_ref[...] = tv
    ti_ref[...] = jnp.concatenate(idxs, axis=1)
    red_ref[...] = jnp.sum(tv, keepdims=True).reshape(1, 1) * (1.0 / _BATCH)


def _tc_merge(prompt_key, sh, st):
    return pl.pallas_call(
        _tc_merge_body,
        grid=(1,),
        in_specs=[
            pl.BlockSpec((_KEYS_PAD, _EMBED), lambda c: (0, 0)),
            pl.BlockSpec((_BATCH, _EMBED), lambda c: (0, 0)),
            pl.BlockSpec((_SPLIT, _BATCH, _EMBED), lambda c: (0, 0, 0)),
        ],
        out_specs=[
            pl.BlockSpec((_BATCH, _NKEYS), lambda c: (0, 0)),
            pl.BlockSpec((_BATCH, _TOPK), lambda c: (0, 0)),
            pl.BlockSpec((_BATCH, _TOPK), lambda c: (0, 0)),
            pl.BlockSpec((1, 1), lambda c: (0, 0)),
        ],
        out_shape=[
            jax.ShapeDtypeStruct((_BATCH, _NKEYS), jnp.float32),
            jax.ShapeDtypeStruct((_BATCH, _TOPK), jnp.float32),
            jax.ShapeDtypeStruct((_BATCH, _TOPK), jnp.int32),
            jax.ShapeDtypeStruct((1, 1), jnp.float32),
        ],
    )(prompt_key, sh, st)


def kernel(x_embed, y, task_id, prompt_key):
    st = _sc_tail_sums(x_embed)
    sh = _tc_head_sums(x_embed)
    sim, topk_sim, topk_idx, red = _tc_merge(prompt_key, sh, st)
    return (sim, topk_sim, topk_idx, red.reshape(()))


# speedup vs baseline: 2.8915x; 1.0198x over previous
import functools

import jax
import jax.numpy as jnp
from jax import lax
from jax.experimental import pallas as pl
from jax.experimental.pallas import tpu as pltpu
from jax.experimental.pallas import tpu_sc as plsc

_EMBED = 768
_SEQ = 2048
_BATCH = 16
_NKEYS = 100
_KEYS_PAD = 104
_TOPK = 3

_TAIL = 64
_HEAD = _SEQ - _TAIL
_NTILES = 32
_SPLIT = 2
_ROWS_PER_TILE = _TAIL // _SPLIT
_CHUNK_ROWS = min(64, _ROWS_PER_TILE)
_NCHUNKS = _ROWS_PER_TILE // _CHUNK_ROWS
_VPR = _EMBED // 16


def _tc_head_body(x_ref, out_ref):
    c = pl.program_id(0)
    out_ref[pl.ds(c, 1), :] = jnp.sum(x_ref[0], axis=0, keepdims=True)


def _tc_head_sums(x):
    return pl.pallas_call(
        _tc_head_body,
        grid=(_BATCH,),
        in_specs=[pl.BlockSpec((1, _HEAD, _EMBED), lambda c: (c, 0, 0))],
        out_specs=pl.BlockSpec((_BATCH, _EMBED), lambda c: (0, 0)),
        out_shape=jax.ShapeDtypeStruct((_BATCH, _EMBED), jnp.float32),
        compiler_params=pltpu.CompilerParams(
            dimension_semantics=("arbitrary",)),
    )(x)


def _sc_tail_sums(x):
    mesh = plsc.VectorSubcoreMesh(core_axis_name="c", subcore_axis_name="s")
    nc = plsc.get_sparse_core_info().num_cores

    @functools.partial(
        pl.kernel,
        mesh=mesh,
        out_type=jax.ShapeDtypeStruct((_SPLIT, _BATCH, _EMBED), jnp.float32),
        scratch_types=[
            pltpu.VMEM((_CHUNK_ROWS, _EMBED), jnp.float32),
            pltpu.VMEM((_CHUNK_ROWS, _EMBED), jnp.float32),
            pltpu.VMEM((_EMBED,), jnp.float32),
            pltpu.SemaphoreType.DMA,
            pltpu.SemaphoreType.DMA,
        ],
    )
    def run(x_hbm, st_hbm, buf0, buf1, acc_v, sem0, sem1):
        wid = lax.axis_index("s") * nc + lax.axis_index("c")
        b = wid // _SPLIT
        h = wid % _SPLIT
        row0 = _HEAD + h * _ROWS_PER_TILE

        bufs = (buf0, buf1)
        sems = (sem0, sem1)

        def start(g):
            return pltpu.async_copy(
                x_hbm.at[b, pl.ds(row0 + g * _CHUNK_ROWS, _CHUNK_ROWS), :],
                bufs[g % 2], sems[g % 2])

        def accum(buf, accs, lo):
            def body(r, carry):
                return tuple(
                    carry[j] + buf[r, pl.ds((lo + j) * 16, 16)]
                    for j in range(len(carry)))
            return lax.fori_loop(0, _CHUNK_ROWS, body, accs)

        half = _VPR // 2
        zero = jnp.zeros((16,), jnp.float32)
        accs_lo = (zero,) * half
        accs_hi = (zero,) * half

        cp = start(0)
        for g in range(_NCHUNKS):
            cp.wait()
            if g + 1 < _NCHUNKS:
                cp = start(g + 1)
            buf = bufs[g % 2]
            accs_lo = accum(buf, accs_lo, 0)
            accs_hi = accum(buf, accs_hi, half)

        for j in range(half):
            acc_v[pl.ds(j * 16, 16)] = accs_lo[j]
            acc_v[pl.ds((half + j) * 16, 16)] = accs_hi[j]
        pltpu.sync_copy(acc_v, st_hbm.at[h, b])

    return run(x)


def _tc_merge_body(pk_ref, sh_ref, st_ref, sim_ref, tv_ref, ti_ref, red_ref):
    total = sh_ref[...] + st_ref[0] + st_ref[1]
    xm = total * (1.0 / _SEQ)
    ss = jnp.sum(xm * xm, axis=-1, keepdims=True)
    xn = xm * lax.rsqrt(jnp.maximum(ss, 1e-12))
    pk = pk_ref[...]
    ps = jnp.sum(pk * pk, axis=-1, keepdims=True)
    pkn = pk * lax.rsqrt(jnp.maximum(ps, 1e-12))
    sim = lax.dot_general(xn, pkn, (((1,), (1,)), ((), ())),
                          preferred_element_type=jnp.float32)
    kiota = lax.broadcasted_iota(jnp.int32, (_BATCH, _KEYS_PAD), 1)
    s = jnp.where(kiota < _NKEYS, sim, -3.0)
    sim_ref[...] = s[:, :_NKEYS]

    vals = []
    idxs = []
    for _ in range(_TOPK):
        m = jnp.max(s, axis=1, keepdims=True)
        i = jnp.min(jnp.where(s == m, kiota, _KEYS_PAD), axis=1,
                    keepdims=True)
        vals.append(m)
        idxs.append(i)
        s = jnp.where(kiota == i, -4.0, s)
    tv = jnp.concatenate(vals, axis=1)
    tv_ref[...] = tv
    ti_ref[...] = jnp.concatenate(idxs, axis=1)
    red_ref[...] = jnp.sum(tv, keepdims=True).reshape(1, 1) * (1.0 / _BATCH)


def _tc_merge(prompt_key, sh, st):
    return pl.pallas_call(
        _tc_merge_body,
        grid=(1,),
        in_specs=[
            pl.BlockSpec((_KEYS_PAD, _EMBED), lambda c: (0, 0)),
            pl.BlockSpec((_BATCH, _EMBED), lambda c: (0, 0)),
            pl.BlockSpec((_SPLIT, _BATCH, _EMBED), lambda c: (0, 0, 0)),
        ],
        out_specs=[
            pl.BlockSpec((_BATCH, _NKEYS), lambda c: (0, 0)),
            pl.BlockSpec((_BATCH, _TOPK), lambda c: (0, 0)),
            pl.BlockSpec((_BATCH, _TOPK), lambda c: (0, 0)),
            pl.BlockSpec((1, 1), lambda c: (0, 0)),
        ],
        out_shape=[
            jax.ShapeDtypeStruct((_BATCH, _NKEYS), jnp.float32),
            jax.ShapeDtypeStruct((_BATCH, _TOPK), jnp.float32),
            jax.ShapeDtypeStruct((_BATCH, _TOPK), jnp.int32),
            jax.ShapeDtypeStruct((1, 1), jnp.float32),
        ],
    )(prompt_key, sh, st)


def kernel(x_embed, y, task_id, prompt_key):
    sh = _tc_head_sums(x_embed)
    st = _sc_tail_sums(x_embed)
    sim, topk_sim, topk_idx, red = _tc_merge(prompt_key, sh, st)
    return (sim, topk_sim, topk_idx, red.reshape(()))
